# baseline (speedup 1.0000x reference)
import functools

import jax
import jax.numpy as jnp
from jax import lax
from jax.experimental import pallas as pl
from jax.experimental.pallas import tpu as pltpu
from jax.experimental.pallas import tpu_sc as plsc

N = 10000
F = 128
E = 320000
NC = 2
NS = 16
NW = NC * NS
EPT = E // NW
CW = 80
NCK_DEG = EPT // CW
GRP = 8
NGRP = 16
NCK_PAD = (NGRP + 2) * GRP
EPT_P = NCK_PAD * CW
NP = 10240
RPT = NP // NS
TRASH = NP - 8
DEGW = 16
DEGWIN = 8

_mesh = plsc.VectorSubcoreMesh(core_axis_name="c", subcore_axis_name="s")


@functools.partial(
    pl.kernel,
    out_type=jax.ShapeDtypeStruct((NC, NP, DEGW), jnp.float32),
    mesh=_mesh,
    scratch_types=[
        pltpu.VMEM((NCK_DEG + 3, CW), jnp.int32),
        pltpu.VMEM((CW, DEGW), jnp.float32),
        pltpu.VMEM_SHARED((NP, DEGW), jnp.float32),
        pltpu.SemaphoreType.DMA,
    ],
)
def _deg_kernel(dst_hbm, out_hbm, idx_v, ones_v, deg_sh, sem):
    cid = lax.axis_index("c")
    sid = lax.axis_index("s")
    wid = sid * NC + cid
    pltpu.sync_copy(dst_hbm.at[wid, pl.ds(0, NCK_DEG + 3)], idx_v)
    ones16 = jnp.ones((DEGW,), jnp.float32)
    zeros16 = jnp.zeros((DEGW,), jnp.float32)

    def _fill_zero(i, _):
        ones_v[i, :] = zeros16
        return 0

    lax.fori_loop(0, CW, _fill_zero, 0)
    for j in range(RPT // CW):
        pltpu.sync_copy(ones_v, deg_sh.at[pl.ds(sid * RPT + j * CW, CW)])

    def _fill_ones(i, _):
        ones_v[i, :] = ones16
        return 0

    lax.fori_loop(0, CW, _fill_ones, 0)
    plsc.subcore_barrier()

    def _fire(c, _):
        pltpu.async_copy(ones_v, deg_sh.at[idx_v.at[c]], sem, add=True)

        @pl.when(c >= DEGWIN)
        def _():
            pltpu.make_async_copy(
                ones_v, deg_sh.at[idx_v.at[c - DEGWIN]], sem).wait()

        return 0

    lax.fori_loop(0, NCK_DEG, _fire, 0)

    def _drain(c, _):
        pltpu.make_async_copy(ones_v, deg_sh.at[idx_v.at[c]], sem).wait()
        return 0

    lax.fori_loop(NCK_DEG - DEGWIN, NCK_DEG, _drain, 0)
    plsc.subcore_barrier()
    pltpu.sync_copy(deg_sh.at[pl.ds(sid * RPT, RPT)],
                    out_hbm.at[cid, pl.ds(sid * RPT, RPT)])


@functools.partial(
    pl.kernel,
    out_type=jax.ShapeDtypeStruct((NC, NP, F), jnp.float32),
    mesh=_mesh,
    scratch_types=[
        pltpu.VMEM((GRP, CW), jnp.int32),
        pltpu.VMEM((GRP, CW), jnp.int32),
        pltpu.VMEM((GRP, CW), jnp.int32),
        pltpu.VMEM((GRP, CW), jnp.int32),
        pltpu.VMEM((CW, F), jnp.float32),
        pltpu.VMEM((CW, F), jnp.float32),
        pltpu.VMEM_SHARED((NP, F), jnp.float32),
        pltpu.SemaphoreType.DMA,
        pltpu.SemaphoreType.DMA,
        pltpu.SemaphoreType.DMA,
        pltpu.SemaphoreType.DMA,
    ],
)
def _scatter_kernel(g_hbm, src_hbm, dst_hbm, out_hbm,
                    sixA, dixA, sixB, dixB, rows0, rows1, acc_sh,
                    semIA, semIB, semG0, semG1):
    cid = lax.axis_index("c")
    sid = lax.axis_index("s")
    wid = sid * NC + cid
    rows = (rows0, rows1)
    semG = (semG0, semG1)

    zeros16 = jnp.zeros((16,), jnp.float32)

    def _fill_zero(k, _):
        i = k // (F // 16)
        j = k % (F // 16)
        rows0[i, pl.ds(j * 16, 16)] = zeros16
        return 0

    lax.fori_loop(0, CW * (F // 16), _fill_zero, 0)
    for j in range(RPT // CW):
        pltpu.sync_copy(rows0, acc_sh.at[pl.ds(sid * RPT + j * CW, CW)])
    plsc.subcore_barrier()

    def _idxcpy(g, six, dix, semI):
        st = pl.multiple_of(g * GRP, GRP)
        pltpu.async_copy(src_hbm.at[wid, pl.ds(st, GRP)], six, semI)
        pltpu.async_copy(dst_hbm.at[wid, pl.ds(st, GRP)], dix, semI)

    def _iwait(g, six, dix, semI):
        st = pl.multiple_of(g * GRP, GRP)
        pltpu.make_async_copy(src_hbm.at[wid, pl.ds(st, GRP)], six, semI).wait()
        pltpu.make_async_copy(dst_hbm.at[wid, pl.ds(st, GRP)], dix, semI).wait()

    def _group(g, six, dix, six_n, dix_n, semI_n, g_pf, six_pf, dix_pf, semI_pf):
        for k in range(GRP):
            rb, sb = rows[k % 2], semG[k % 2]
            rb_n, sb_n = rows[(k + 1) % 2], semG[(k + 1) % 2]
            if k < GRP - 1:
                pltpu.async_copy(g_hbm.at[six.at[k + 1]], rb_n, sb_n)
            else:
                _iwait(g + 1, six_n, dix_n, semI_n)
                pltpu.async_copy(g_hbm.at[six_n.at[0]], rb_n, sb_n)
            pltpu.make_async_copy(g_hbm.at[six.at[k]], rb, sb).wait()
            pltpu.sync_copy(rb, acc_sh.at[dix.at[k]], add=True)
            if k == GRP - 1:
                _idxcpy(g_pf, six_pf, dix_pf, semI_pf)

    _idxcpy(0, sixA, dixA, semIA)
    _idxcpy(1, sixB, dixB, semIB)
    _iwait(0, sixA, dixA, semIA)
    pltpu.async_copy(g_hbm.at[sixA.at[0]], rows0, semG0)

    def _pair(i, _):
        gA = 2 * i
        _group(gA, sixA, dixA, sixB, dixB, semIB,
               gA + 2, sixA, dixA, semIA)
        _group(gA + 1, sixB, dixB, sixA, dixA, semIA,
               gA + 3, sixB, dixB, semIB)
        return 0

    lax.fori_loop(0, NGRP // 2, _pair, 0)
    pltpu.make_async_copy(g_hbm.at[sixA.at[0]], rows0, semG0).wait()
    _iwait(NGRP + 1, sixB, dixB, semIB)
    plsc.subcore_barrier()
    pltpu.sync_copy(acc_sh.at[pl.ds(sid * RPT, RPT)],
                    out_hbm.at[cid, pl.ds(sid * RPT, RPT)])


def _mid_body(x_ref, wi_ref, bi_ref, wg_ref, d0_ref, d1_ref, g_ref, dinv_ref):
    h = jnp.maximum(
        jnp.dot(x_ref[...], wi_ref[...], preferred_element_type=jnp.float32)
        + bi_ref[...], 0.0)
    hw = jnp.dot(h, wg_ref[...], preferred_element_type=jnp.float32)
    deg = d0_ref[...] + d1_ref[...] + 1.0
    dinv = lax.rsqrt(deg)
    g_ref[...] = hw * dinv
    dinv_ref[...] = dinv


def _out_body(a_ref, g_ref, dinv_ref, bg_ref, wo_ref, bo_ref, o_ref):
    pre = (a_ref[0] + a_ref[1] + g_ref[...]) * dinv_ref[...] + bg_ref[...]
    o_ref[...] = jnp.dot(jnp.maximum(pre, 0.0), wo_ref[...],
                         preferred_element_type=jnp.float32) + bo_ref[...]


N_BLK = 1000


def kernel(x, edge_index, W_in, b_in, W_gcn, b_gcn, W_out, b_out):
    nclass = W_out.shape[0]
    src2d = edge_index[0].astype(jnp.int32).reshape(NW, EPT)
    dst2d = edge_index[1].astype(jnp.int32).reshape(NW, EPT)
    npad = EPT_P - EPT
    src_p = jnp.concatenate(
        [src2d, jnp.zeros((NW, npad), jnp.int32)], axis=1
    ).reshape(NW, NCK_PAD, CW)
    dst_p = jnp.concatenate(
        [dst2d, jnp.full((NW, npad), TRASH, jnp.int32)], axis=1
    ).reshape(NW, NCK_PAD, CW)

    degp = _deg_kernel(dst_p)
    d0 = degp[0, :N, 0:1]
    d1 = degp[1, :N, 0:1]

    g, dinv = pl.pallas_call(
        _mid_body,
        grid=(N // N_BLK,),
        in_specs=[
            pl.BlockSpec((N_BLK, F), lambda i: (i, 0)),
            pl.BlockSpec((F, F), lambda i: (0, 0)),
            pl.BlockSpec((1, F), lambda i: (0, 0)),
            pl.BlockSpec((F, F), lambda i: (0, 0)),
            pl.BlockSpec((N_BLK, 1), lambda i: (i, 0)),
            pl.BlockSpec((N_BLK, 1), lambda i: (i, 0)),
        ],
        out_specs=[
            pl.BlockSpec((N_BLK, F), lambda i: (i, 0)),
            pl.BlockSpec((N_BLK, 1), lambda i: (i, 0)),
        ],
        out_shape=[
            jax.ShapeDtypeStruct((N, F), jnp.float32),
            jax.ShapeDtypeStruct((N, 1), jnp.float32),
        ],
    )(x, W_in.T, b_in.reshape(1, F), W_gcn.T, d0, d1)

    acc = _scatter_kernel(g, src_p, dst_p)[:, :N, :]

    y = pl.pallas_call(
        _out_body,
        grid=(N // N_BLK,),
        in_specs=[
            pl.BlockSpec((NC, N_BLK, F), lambda i: (0, i, 0)),
            pl.BlockSpec((N_BLK, F), lambda i: (i, 0)),
            pl.BlockSpec((N_BLK, 1), lambda i: (i, 0)),
            pl.BlockSpec((1, F), lambda i: (0, 0)),
            pl.BlockSpec((F, nclass), lambda i: (0, 0)),
            pl.BlockSpec((1, nclass), lambda i: (0, 0)),
        ],
        out_specs=pl.BlockSpec((N_BLK, nclass), lambda i: (i, 0)),
        out_shape=jax.ShapeDtypeStruct((N, nclass), jnp.float32),
    )(acc, g, dinv, b_gcn.reshape(1, F), W_out.T, b_out.reshape(1, nclass))
    return y


# speedup vs baseline: 1.1621x; 1.1621x over previous
import functools

import jax
import jax.numpy as jnp
from jax import lax
from jax.experimental import pallas as pl
from jax.experimental.pallas import tpu as pltpu
from jax.experimental.pallas import tpu_sc as plsc

N = 10000
F = 128
E = 320000
NC = 2
NS = 16
NW = NC * NS
EPT = E // NW
CW = 80
NCK_DEG = EPT // CW
GRP = 8
NGRP = 16
NCK_PAD = (NGRP + 2) * GRP
EPT_P = NCK_PAD * CW
NP = 10240
RPT = NP // NS
TRASH = NP - 8
DEGW = 16
DEGWIN = 8

_mesh = plsc.VectorSubcoreMesh(core_axis_name="c", subcore_axis_name="s")


@functools.partial(
    pl.kernel,
    out_type=jax.ShapeDtypeStruct((NC, NP, DEGW), jnp.float32),
    mesh=_mesh,
    scratch_types=[
        pltpu.VMEM((NCK_DEG + 3, CW), jnp.int32),
        pltpu.VMEM((CW, DEGW), jnp.float32),
        pltpu.VMEM_SHARED((NP, DEGW), jnp.float32),
        pltpu.SemaphoreType.DMA,
    ],
)
def _deg_kernel(dst_hbm, out_hbm, idx_v, ones_v, deg_sh, sem):
    cid = lax.axis_index("c")
    sid = lax.axis_index("s")
    wid = sid * NC + cid
    pltpu.sync_copy(dst_hbm.at[wid, pl.ds(0, NCK_DEG + 3)], idx_v)
    ones16 = jnp.ones((DEGW,), jnp.float32)
    zeros16 = jnp.zeros((DEGW,), jnp.float32)

    def _fill_zero(i, _):
        ones_v[i, :] = zeros16
        return 0

    lax.fori_loop(0, CW, _fill_zero, 0)
    for j in range(RPT // CW):
        pltpu.sync_copy(ones_v, deg_sh.at[pl.ds(sid * RPT + j * CW, CW)])

    def _fill_ones(i, _):
        ones_v[i, :] = ones16
        return 0

    lax.fori_loop(0, CW, _fill_ones, 0)
    plsc.subcore_barrier()

    def _fire(c, _):
        pltpu.async_copy(ones_v, deg_sh.at[idx_v.at[c]], sem, add=True)

        @pl.when(c >= DEGWIN)
        def _():
            pltpu.make_async_copy(
                ones_v, deg_sh.at[idx_v.at[c - DEGWIN]], sem).wait()

        return 0

    lax.fori_loop(0, NCK_DEG, _fire, 0)

    def _drain(c, _):
        pltpu.make_async_copy(ones_v, deg_sh.at[idx_v.at[c]], sem).wait()
        return 0

    lax.fori_loop(NCK_DEG - DEGWIN, NCK_DEG, _drain, 0)
    plsc.subcore_barrier()
    pltpu.sync_copy(deg_sh.at[pl.ds(sid * RPT, RPT)],
                    out_hbm.at[cid, pl.ds(sid * RPT, RPT)])


@functools.partial(
    pl.kernel,
    out_type=jax.ShapeDtypeStruct((NC, NP, F), jnp.float32),
    mesh=_mesh,
    scratch_types=[
        pltpu.VMEM((GRP, CW), jnp.int32),
        pltpu.VMEM((GRP, CW), jnp.int32),
        pltpu.VMEM((GRP, CW), jnp.int32),
        pltpu.VMEM((GRP, CW), jnp.int32),
        pltpu.VMEM((CW, F), jnp.float32),
        pltpu.VMEM((CW, F), jnp.float32),
        pltpu.VMEM_SHARED((NP, F), jnp.float32),
        pltpu.SemaphoreType.DMA,
        pltpu.SemaphoreType.DMA,
        pltpu.SemaphoreType.DMA,
        pltpu.SemaphoreType.DMA,
    ],
)
def _scatter_kernel(g_hbm, src_hbm, dst_hbm, out_hbm,
                    sixA, dixA, sixB, dixB, rows0, rows1, acc_sh,
                    semIA, semIB, semG0, semG1):
    cid = lax.axis_index("c")
    sid = lax.axis_index("s")
    wid = sid * NC + cid
    rows = (rows0, rows1)
    semG = (semG0, semG1)

    zeros16 = jnp.zeros((16,), jnp.float32)

    def _fill_zero(k, _):
        i = k // (F // 16)
        j = k % (F // 16)
        rows0[i, pl.ds(j * 16, 16)] = zeros16
        return 0

    lax.fori_loop(0, CW * (F // 16), _fill_zero, 0)
    for j in range(RPT // CW):
        pltpu.sync_copy(rows0, acc_sh.at[pl.ds(sid * RPT + j * CW, CW)])
    plsc.subcore_barrier()

    def _idxcpy(g, six, dix, semI):
        st = pl.multiple_of(g * GRP, GRP)
        pltpu.async_copy(src_hbm.at[wid, pl.ds(st, GRP)], six, semI)
        pltpu.async_copy(dst_hbm.at[wid, pl.ds(st, GRP)], dix, semI)

    def _iwait(g, six, dix, semI):
        st = pl.multiple_of(g * GRP, GRP)
        pltpu.make_async_copy(src_hbm.at[wid, pl.ds(st, GRP)], six, semI).wait()
        pltpu.make_async_copy(dst_hbm.at[wid, pl.ds(st, GRP)], dix, semI).wait()

    def _group(g, six, dix, semI, g_pf):
        _iwait(g, six, dix, semI)
        cps = [pltpu.async_copy(g_hbm.at[six.at[0]], rows0, semG0)]
        for k in range(GRP):
            if k + 1 < GRP:
                cps.append(pltpu.async_copy(
                    g_hbm.at[six.at[k + 1]], rows[(k + 1) % 2],
                    semG[(k + 1) % 2]))
            cps[k].wait()
            pltpu.sync_copy(rows[k % 2], acc_sh.at[dix.at[k]], add=True)
            if k == GRP - 1:
                _idxcpy(g_pf, six, dix, semI)

    _idxcpy(0, sixA, dixA, semIA)
    _idxcpy(1, sixB, dixB, semIB)

    def _pair(i, _):
        gA = 2 * i
        _group(gA, sixA, dixA, semIA, gA + 2)
        _group(gA + 1, sixB, dixB, semIB, gA + 3)
        return 0

    lax.fori_loop(0, NGRP // 2, _pair, 0)
    _iwait(NGRP, sixA, dixA, semIA)
    _iwait(NGRP + 1, sixB, dixB, semIB)
    plsc.subcore_barrier()
    pltpu.sync_copy(acc_sh.at[pl.ds(sid * RPT, RPT)],
                    out_hbm.at[cid, pl.ds(sid * RPT, RPT)])


def _mid_body(x_ref, wi_ref, bi_ref, wg_ref, d0_ref, d1_ref, g_ref, dinv_ref):
    h = jnp.maximum(
        jnp.dot(x_ref[...], wi_ref[...], preferred_element_type=jnp.float32)
        + bi_ref[...], 0.0)
    hw = jnp.dot(h, wg_ref[...], preferred_element_type=jnp.float32)
    deg = d0_ref[...] + d1_ref[...] + 1.0
    dinv = lax.rsqrt(deg)
    g_ref[...] = hw * dinv
    dinv_ref[...] = dinv


def _out_body(a_ref, g_ref, dinv_ref, bg_ref, wo_ref, bo_ref, o_ref):
    pre = (a_ref[0] + a_ref[1] + g_ref[...]) * dinv_ref[...] + bg_ref[...]
    o_ref[...] = jnp.dot(jnp.maximum(pre, 0.0), wo_ref[...],
                         preferred_element_type=jnp.float32) + bo_ref[...]


N_BLK = 1000


def kernel(x, edge_index, W_in, b_in, W_gcn, b_gcn, W_out, b_out):
    nclass = W_out.shape[0]
    src2d = edge_index[0].astype(jnp.int32).reshape(NW, EPT)
    dst2d = edge_index[1].astype(jnp.int32).reshape(NW, EPT)
    npad = EPT_P - EPT
    src_p = jnp.concatenate(
        [src2d, jnp.zeros((NW, npad), jnp.int32)], axis=1
    ).reshape(NW, NCK_PAD, CW)
    dst_p = jnp.concatenate(
        [dst2d, jnp.full((NW, npad), TRASH, jnp.int32)], axis=1
    ).reshape(NW, NCK_PAD, CW)

    degp = _deg_kernel(dst_p)
    d0 = degp[0, :N, 0:1]
    d1 = degp[1, :N, 0:1]

    g, dinv = pl.pallas_call(
        _mid_body,
        grid=(N // N_BLK,),
        in_specs=[
            pl.BlockSpec((N_BLK, F), lambda i: (i, 0)),
            pl.BlockSpec((F, F), lambda i: (0, 0)),
            pl.BlockSpec((1, F), lambda i: (0, 0)),
            pl.BlockSpec((F, F), lambda i: (0, 0)),
            pl.BlockSpec((N_BLK, 1), lambda i: (i, 0)),
            pl.BlockSpec((N_BLK, 1), lambda i: (i, 0)),
        ],
        out_specs=[
            pl.BlockSpec((N_BLK, F), lambda i: (i, 0)),
            pl.BlockSpec((N_BLK, 1), lambda i: (i, 0)),
        ],
        out_shape=[
            jax.ShapeDtypeStruct((N, F), jnp.float32),
            jax.ShapeDtypeStruct((N, 1), jnp.float32),
        ],
    )(x, W_in.T, b_in.reshape(1, F), W_gcn.T, d0, d1)

    acc = _scatter_kernel(g, src_p, dst_p)[:, :N, :]

    y = pl.pallas_call(
        _out_body,
        grid=(N // N_BLK,),
        in_specs=[
            pl.BlockSpec((NC, N_BLK, F), lambda i: (0, i, 0)),
            pl.BlockSpec((N_BLK, F), lambda i: (i, 0)),
            pl.BlockSpec((N_BLK, 1), lambda i: (i, 0)),
            pl.BlockSpec((1, F), lambda i: (0, 0)),
            pl.BlockSpec((F, nclass), lambda i: (0, 0)),
            pl.BlockSpec((1, nclass), lambda i: (0, 0)),
        ],
        out_specs=pl.BlockSpec((N_BLK, nclass), lambda i: (i, 0)),
        out_shape=jax.ShapeDtypeStruct((N, nclass), jnp.float32),
    )(acc, g, dinv, b_gcn.reshape(1, F), W_out.T, b_out.reshape(1, nclass))
    return y
